# baseline (speedup 1.0000x reference)
import functools

import jax
import jax.numpy as jnp
from jax import lax
from jax.experimental import pallas as pl
from jax.experimental.pallas import tpu as pltpu
from jax.experimental.pallas import tpu_sc as plsc

NUM_EXPERTS = 64
TOP_K = 2
D_IN = 768
D_H = 64
D_OUT = 768
T = 2048

TB = 128
N_TB = T // TB
ALIGN = 8
CHUNK = 128
N_BUF = 4736

NC = 2
NS = 16
NW = NC * NS
TPW = T // NW
LANES = 16


def _routing_body(x_ref, wg_ref, bg_ref,
                  pos0_ref, pos1_ref, w0_ref, w1_ref, ce_ref, cb_ref,
                  oh0_s, oh1_s, r0_s, r1_s, w0_s, w1_s, cnt_s):
    i = pl.program_id(0)

    @pl.when(i == 0)
    def _init():
        cnt_s[...] = jnp.zeros((1, NUM_EXPERTS), jnp.float32)

    @pl.when(i < N_TB)
    def _pass_a():
        xb = x_ref[...]
        logits = jnp.dot(xb, wg_ref[...],
                         preferred_element_type=jnp.float32) + bg_ref[...]
        m = jnp.max(logits, axis=1, keepdims=True)
        e = jnp.exp(logits - m)
        w = e / jnp.sum(e, axis=1, keepdims=True)

        lane = lax.broadcasted_iota(jnp.int32, (TB, NUM_EXPERTS), 1)
        m1 = jnp.max(w, axis=1, keepdims=True)
        i1 = jnp.min(jnp.where(w == m1, lane, NUM_EXPERTS), axis=1,
                     keepdims=True)
        oh0 = (lane == i1).astype(jnp.float32)
        wm = jnp.where(lane == i1, -1.0, w)
        m2 = jnp.max(wm, axis=1, keepdims=True)
        i2 = jnp.min(jnp.where(wm == m2, lane, NUM_EXPERTS), axis=1,
                     keepdims=True)
        oh1 = (lane == i2).astype(jnp.float32)

        s = oh0 + oh1
        row = lax.broadcasted_iota(jnp.int32, (TB, TB), 0)
        col = lax.broadcasted_iota(jnp.int32, (TB, TB), 1)
        ltri = (col < row).astype(jnp.float32)
        prefix = jnp.dot(ltri, s, preferred_element_type=jnp.float32)
        carry = cnt_s[...]
        base = carry + prefix
        r0 = jnp.sum(oh0 * base, axis=1, keepdims=True)
        r1 = jnp.sum(oh1 * (base + oh0), axis=1, keepdims=True)
        cnt_s[...] = carry + jnp.sum(s, axis=0, keepdims=True)

        sl = pl.ds(i * TB, TB)
        oh0_s[sl, :] = oh0
        oh1_s[sl, :] = oh1
        r0_s[sl, :] = r0
        r1_s[sl, :] = r1
        w0_s[sl, :] = m1
        w1_s[sl, :] = m2

    @pl.when(i == N_TB)
    def _finalize():
        cnt = cnt_s[...]
        nal = jnp.floor((cnt + (ALIGN - 1)) * (1.0 / ALIGN)) * float(ALIGN)
        erow = lax.broadcasted_iota(jnp.int32, (NUM_EXPERTS, NUM_EXPERTS), 0)
        ecol = lax.broadcasted_iota(jnp.int32, (NUM_EXPERTS, NUM_EXPERTS), 1)
        nal_b = jnp.broadcast_to(nal, (NUM_EXPERTS, NUM_EXPERTS))
        offs_c = jnp.sum(jnp.where(ecol < erow, nal_b, 0.0),
                         axis=1, keepdims=True)
        offs_row = jnp.sum(
            jnp.where(erow == ecol,
                      jnp.broadcast_to(offs_c, (NUM_EXPERTS, NUM_EXPERTS)),
                      0.0), axis=0, keepdims=True)

        pos0 = r0_s[...] + jnp.sum(oh0_s[...] * offs_row, axis=1,
                                   keepdims=True)
        pos1 = r1_s[...] + jnp.sum(oh1_s[...] * offs_row, axis=1,
                                   keepdims=True)
        pos0_ref[...] = pos0.astype(jnp.int32)
        pos1_ref[...] = pos1.astype(jnp.int32)
        w0_ref[...] = w0_s[...]
        w1_ref[...] = w1_s[...]
        ce_ref[...] = offs_row.astype(jnp.int32)
        cb_ref[...] = cnt.astype(jnp.int32)


def _routing(x, Wg, bg):
    f32 = jnp.float32
    i32 = jnp.int32
    return pl.pallas_call(
        _routing_body,
        grid=(N_TB + 1,),
        in_specs=[
            pl.BlockSpec((TB, D_IN), lambda i: (jnp.minimum(i, N_TB - 1), 0)),
            pl.BlockSpec((D_IN, NUM_EXPERTS), lambda i: (0, 0)),
            pl.BlockSpec((1, NUM_EXPERTS), lambda i: (0, 0)),
        ],
        out_specs=[
            pl.BlockSpec((T, 1), lambda i: (0, 0)),
            pl.BlockSpec((T, 1), lambda i: (0, 0)),
            pl.BlockSpec((T, 1), lambda i: (0, 0)),
            pl.BlockSpec((T, 1), lambda i: (0, 0)),
            pl.BlockSpec((1, NUM_EXPERTS), lambda i: (0, 0)),
            pl.BlockSpec((1, NUM_EXPERTS), lambda i: (0, 0)),
        ],
        out_shape=[
            jax.ShapeDtypeStruct((T, 1), i32),
            jax.ShapeDtypeStruct((T, 1), i32),
            jax.ShapeDtypeStruct((T, 1), f32),
            jax.ShapeDtypeStruct((T, 1), f32),
            jax.ShapeDtypeStruct((1, NUM_EXPERTS), i32),
            jax.ShapeDtypeStruct((1, NUM_EXPERTS), i32),
        ],
        scratch_shapes=[
            pltpu.VMEM((T, NUM_EXPERTS), f32),
            pltpu.VMEM((T, NUM_EXPERTS), f32),
            pltpu.VMEM((T, 1), f32),
            pltpu.VMEM((T, 1), f32),
            pltpu.VMEM((T, 1), f32),
            pltpu.VMEM((T, 1), f32),
            pltpu.VMEM((1, NUM_EXPERTS), f32),
        ],
        compiler_params=pltpu.CompilerParams(
            dimension_semantics=("arbitrary",)),
    )(x, Wg, bg.reshape(1, NUM_EXPERTS))


def _dispatch_body(x_hbm, pos_hbm, xs_hbm, pos_v, rows_v, sem0, sem1):
    wid = lax.axis_index("s") * NC + lax.axis_index("c")
    base = wid * TPW
    pltpu.sync_copy(pos_hbm.at[wid], pos_v)
    pltpu.sync_copy(x_hbm.at[pl.ds(base, TPW)], rows_v)
    c0 = pltpu.async_copy(rows_v, xs_hbm.at[pos_v.at[0]], sem0)
    c1 = pltpu.async_copy(rows_v, xs_hbm.at[pos_v.at[1]], sem1)
    c0.wait()
    c1.wait()


def _dispatch(x, pos3):
    mesh = plsc.VectorSubcoreMesh(core_axis_name="c", subcore_axis_name="s")
    fn = pl.kernel(
        _dispatch_body,
        out_type=jax.ShapeDtypeStruct((N_BUF, D_IN), jnp.float32),
        mesh=mesh,
        scratch_types=[
            pltpu.VMEM((TOP_K, TPW), jnp.int32),
            pltpu.VMEM((TPW, D_IN), jnp.float32),
            pltpu.SemaphoreType.DMA,
            pltpu.SemaphoreType.DMA,
        ],
    )
    return fn(x, pos3)


def _ffn_body(offs_ref, cnt_ref, xs_ref, w1_ref, b1_ref, w2_ref, b2_ref,
              ys_ref):
    e = pl.program_id(0)
    off = offs_ref[e]
    nch = (cnt_ref[e] + (CHUNK - 1)) // CHUNK

    w1 = w1_ref[0]
    b1v = b1_ref[0]
    w2 = w2_ref[0]
    b2v = b2_ref[0]

    cn = cnt_ref[e]
    row = lax.broadcasted_iota(jnp.int32, (CHUNK, D_OUT), 0)

    def chunk_body(j, carry):
        sl = pl.ds(pl.multiple_of(off + j * CHUNK, ALIGN), CHUNK)
        xb = xs_ref[sl, :]
        h = jnp.maximum(
            jnp.dot(xb, w1, preferred_element_type=jnp.float32) + b1v, 0.0)
        y = jnp.dot(h, w2, preferred_element_type=jnp.float32) + b2v
        ys_ref[sl, :] = jnp.where(row < cn - j * CHUNK, y, ys_ref[sl, :])
        return carry

    lax.fori_loop(0, nch, chunk_body, 0)


def _ffn(offs, cnt, xs, W1, b1, W2, b2):
    grid_spec = pltpu.PrefetchScalarGridSpec(
        num_scalar_prefetch=2,
        grid=(NUM_EXPERTS,),
        in_specs=[
            pl.BlockSpec((N_BUF, D_IN), lambda e, offs, cnt: (0, 0)),
            pl.BlockSpec((1, D_IN, D_H), lambda e, offs, cnt: (e, 0, 0)),
            pl.BlockSpec((1, 1, D_H), lambda e, offs, cnt: (e, 0, 0)),
            pl.BlockSpec((1, D_H, D_OUT), lambda e, offs, cnt: (e, 0, 0)),
            pl.BlockSpec((1, 1, D_OUT), lambda e, offs, cnt: (e, 0, 0)),
        ],
        out_specs=pl.BlockSpec((N_BUF, D_OUT), lambda e, offs, cnt: (0, 0)),
    )
    return pl.pallas_call(
        _ffn_body,
        grid_spec=grid_spec,
        out_shape=jax.ShapeDtypeStruct((N_BUF, D_OUT), jnp.float32),
        compiler_params=pltpu.CompilerParams(
            dimension_semantics=("arbitrary",)),
    )(offs, cnt, xs, W1, b1.reshape(NUM_EXPERTS, 1, D_H),
      W2, b2.reshape(NUM_EXPERTS, 1, D_OUT))


def _combine_body(ys_hbm, pos_hbm, wv_hbm, out_hbm,
                  pos_v, wv_v, y0_v, y1_v, sem0, sem1):
    wid = lax.axis_index("s") * NC + lax.axis_index("c")
    base = wid * TPW
    pltpu.sync_copy(pos_hbm.at[wid], pos_v)
    pltpu.sync_copy(wv_hbm.at[wid], wv_v)
    g0 = pltpu.async_copy(ys_hbm.at[pos_v.at[0]], y0_v, sem0)
    g1 = pltpu.async_copy(ys_hbm.at[pos_v.at[1]], y1_v, sem1)
    g0.wait()
    g1.wait()

    def tbody(t, carry):
        w0 = wv_v[0, t, :]
        w1 = wv_v[1, t, :]
        for c in range(D_OUT // LANES):
            sl = pl.ds(c * LANES, LANES)
            y0_v[t, sl] = y0_v[t, sl] * w0 + y1_v[t, sl] * w1
        return carry

    lax.fori_loop(0, TPW, tbody, 0)
    pltpu.sync_copy(y0_v, out_hbm.at[pl.ds(base, TPW)])


def _combine(ys, pos3, wv3):
    mesh = plsc.VectorSubcoreMesh(core_axis_name="c", subcore_axis_name="s")
    fn = pl.kernel(
        _combine_body,
        out_type=jax.ShapeDtypeStruct((T, D_OUT), jnp.float32),
        mesh=mesh,
        scratch_types=[
            pltpu.VMEM((TOP_K, TPW), jnp.int32),
            pltpu.VMEM((TOP_K, TPW, LANES), jnp.float32),
            pltpu.VMEM((TPW, D_OUT), jnp.float32),
            pltpu.VMEM((TPW, D_OUT), jnp.float32),
            pltpu.SemaphoreType.DMA,
            pltpu.SemaphoreType.DMA,
        ],
    )
    return fn(ys, pos3, wv3)


def kernel(x, Wg, bg, W1, b1, W2, b2):
    pos0, pos1, w0, w1, offs, cnt = _routing(x, Wg, bg)

    pos3 = jnp.stack([pos0.reshape(NW, TPW), pos1.reshape(NW, TPW)], axis=1)
    wv3 = jnp.broadcast_to(
        jnp.stack([w0.reshape(NW, TPW), w1.reshape(NW, TPW)], axis=1)[..., None],
        (NW, TOP_K, TPW, LANES))

    xs = _dispatch(x, pos3)
    ys = _ffn(offs.reshape(NUM_EXPERTS), cnt.reshape(NUM_EXPERTS),
              xs, W1, b1, W2, b2)
    return _combine(ys, pos3, wv3)


# speedup vs baseline: 1.0018x; 1.0018x over previous
import functools

import jax
import jax.numpy as jnp
from jax import lax
from jax.experimental import pallas as pl
from jax.experimental.pallas import tpu as pltpu
from jax.experimental.pallas import tpu_sc as plsc

NUM_EXPERTS = 64
TOP_K = 2
D_IN = 768
D_H = 64
D_OUT = 768
T = 2048

TB = 128
N_TB = T // TB
ALIGN = 8
CHUNK = 128
N_BUF = 4736

NC = 2
NS = 16
NW = NC * NS
TPW = T // NW
LANES = 16


def _routing_body(x_ref, wg_ref, bg_ref,
                  pos0_ref, pos1_ref, w0_ref, w1_ref, ce_ref, cb_ref,
                  oh0_s, oh1_s, r0_s, r1_s, w0_s, w1_s, cnt_s):
    i = pl.program_id(0)

    @pl.when(i == 0)
    def _init():
        cnt_s[...] = jnp.zeros((1, NUM_EXPERTS), jnp.float32)

    @pl.when(i < N_TB)
    def _pass_a():
        xb = x_ref[...]
        logits = jnp.dot(xb, wg_ref[...],
                         preferred_element_type=jnp.float32) + bg_ref[...]
        m = jnp.max(logits, axis=1, keepdims=True)
        e = jnp.exp(logits - m)
        w = e / jnp.sum(e, axis=1, keepdims=True)

        lane = lax.broadcasted_iota(jnp.int32, (TB, NUM_EXPERTS), 1)
        m1 = jnp.max(w, axis=1, keepdims=True)
        i1 = jnp.min(jnp.where(w == m1, lane, NUM_EXPERTS), axis=1,
                     keepdims=True)
        oh0 = (lane == i1).astype(jnp.float32)
        wm = jnp.where(lane == i1, -1.0, w)
        m2 = jnp.max(wm, axis=1, keepdims=True)
        i2 = jnp.min(jnp.where(wm == m2, lane, NUM_EXPERTS), axis=1,
                     keepdims=True)
        oh1 = (lane == i2).astype(jnp.float32)

        s = oh0 + oh1
        row = lax.broadcasted_iota(jnp.int32, (TB, TB), 0)
        col = lax.broadcasted_iota(jnp.int32, (TB, TB), 1)
        ltri = (col < row).astype(jnp.float32)
        prefix = jnp.dot(ltri, s, preferred_element_type=jnp.float32)
        carry = cnt_s[...]
        base = carry + prefix
        r0 = jnp.sum(oh0 * base, axis=1, keepdims=True)
        r1 = jnp.sum(oh1 * (base + oh0), axis=1, keepdims=True)
        cnt_s[...] = carry + jnp.sum(s, axis=0, keepdims=True)

        sl = pl.ds(i * TB, TB)
        oh0_s[sl, :] = oh0
        oh1_s[sl, :] = oh1
        r0_s[sl, :] = r0
        r1_s[sl, :] = r1
        w0_s[sl, :] = m1
        w1_s[sl, :] = m2

    @pl.when(i == N_TB)
    def _finalize():
        cnt = cnt_s[...]
        nal = jnp.floor((cnt + (ALIGN - 1)) * (1.0 / ALIGN)) * float(ALIGN)
        erow = lax.broadcasted_iota(jnp.int32, (NUM_EXPERTS, NUM_EXPERTS), 0)
        ecol = lax.broadcasted_iota(jnp.int32, (NUM_EXPERTS, NUM_EXPERTS), 1)
        nal_b = jnp.broadcast_to(nal, (NUM_EXPERTS, NUM_EXPERTS))
        offs_c = jnp.sum(jnp.where(ecol < erow, nal_b, 0.0),
                         axis=1, keepdims=True)
        offs_row = jnp.sum(
            jnp.where(erow == ecol,
                      jnp.broadcast_to(offs_c, (NUM_EXPERTS, NUM_EXPERTS)),
                      0.0), axis=0, keepdims=True)

        pos0 = r0_s[...] + jnp.sum(oh0_s[...] * offs_row, axis=1,
                                   keepdims=True)
        pos1 = r1_s[...] + jnp.sum(oh1_s[...] * offs_row, axis=1,
                                   keepdims=True)
        pos0_ref[...] = pos0.astype(jnp.int32)
        pos1_ref[...] = pos1.astype(jnp.int32)
        w0_ref[...] = w0_s[...]
        w1_ref[...] = w1_s[...]
        ce_ref[...] = offs_row.astype(jnp.int32)
        cb_ref[...] = cnt.astype(jnp.int32)


def _routing(x, Wg, bg):
    f32 = jnp.float32
    i32 = jnp.int32
    return pl.pallas_call(
        _routing_body,
        grid=(N_TB + 1,),
        in_specs=[
            pl.BlockSpec((TB, D_IN), lambda i: (jnp.minimum(i, N_TB - 1), 0)),
            pl.BlockSpec((D_IN, NUM_EXPERTS), lambda i: (0, 0)),
            pl.BlockSpec((1, NUM_EXPERTS), lambda i: (0, 0)),
        ],
        out_specs=[
            pl.BlockSpec((T, 1), lambda i: (0, 0)),
            pl.BlockSpec((T, 1), lambda i: (0, 0)),
            pl.BlockSpec((T, 1), lambda i: (0, 0)),
            pl.BlockSpec((T, 1), lambda i: (0, 0)),
            pl.BlockSpec((1, NUM_EXPERTS), lambda i: (0, 0)),
            pl.BlockSpec((1, NUM_EXPERTS), lambda i: (0, 0)),
        ],
        out_shape=[
            jax.ShapeDtypeStruct((T, 1), i32),
            jax.ShapeDtypeStruct((T, 1), i32),
            jax.ShapeDtypeStruct((T, 1), f32),
            jax.ShapeDtypeStruct((T, 1), f32),
            jax.ShapeDtypeStruct((1, NUM_EXPERTS), i32),
            jax.ShapeDtypeStruct((1, NUM_EXPERTS), i32),
        ],
        scratch_shapes=[
            pltpu.VMEM((T, NUM_EXPERTS), f32),
            pltpu.VMEM((T, NUM_EXPERTS), f32),
            pltpu.VMEM((T, 1), f32),
            pltpu.VMEM((T, 1), f32),
            pltpu.VMEM((T, 1), f32),
            pltpu.VMEM((T, 1), f32),
            pltpu.VMEM((1, NUM_EXPERTS), f32),
        ],
        compiler_params=pltpu.CompilerParams(
            dimension_semantics=("arbitrary",)),
    )(x, Wg, bg.reshape(1, NUM_EXPERTS))


def _dispatch_body(x_hbm, pos_hbm, xs_hbm, pos_v, rows_v, sem0, sem1):
    wid = lax.axis_index("s") * NC + lax.axis_index("c")
    base = wid * TPW
    pltpu.sync_copy(pos_hbm.at[wid], pos_v)
    pltpu.sync_copy(x_hbm.at[pl.ds(base, TPW)], rows_v)
    c0 = pltpu.async_copy(rows_v, xs_hbm.at[pos_v.at[0]], sem0)
    c1 = pltpu.async_copy(rows_v, xs_hbm.at[pos_v.at[1]], sem1)
    c0.wait()
    c1.wait()


def _dispatch(x, pos3):
    mesh = plsc.VectorSubcoreMesh(core_axis_name="c", subcore_axis_name="s")
    fn = pl.kernel(
        _dispatch_body,
        out_type=jax.ShapeDtypeStruct((N_BUF, D_IN), jnp.float32),
        mesh=mesh,
        scratch_types=[
            pltpu.VMEM((TOP_K, TPW), jnp.int32),
            pltpu.VMEM((TPW, D_IN), jnp.float32),
            pltpu.SemaphoreType.DMA,
            pltpu.SemaphoreType.DMA,
        ],
    )
    return fn(x, pos3)


def _ffn_body(offs_ref, cnt_ref, xs_ref, w1_ref, b1_ref, w2_ref, b2_ref,
              ys_ref):
    e = pl.program_id(0)
    off = offs_ref[e]
    nch = (cnt_ref[e] + (CHUNK - 1)) // CHUNK

    w1 = w1_ref[0]
    b1v = b1_ref[0]
    w2 = w2_ref[0]
    b2v = b2_ref[0]

    def chunk_body(j, carry):
        sl = pl.ds(pl.multiple_of(off + j * CHUNK, ALIGN), CHUNK)
        xb = xs_ref[sl, :]
        h = jnp.maximum(
            jnp.dot(xb, w1, preferred_element_type=jnp.float32) + b1v, 0.0)
        y = jnp.dot(h, w2, preferred_element_type=jnp.float32) + b2v
        ys_ref[sl, :] = y
        return carry

    lax.fori_loop(0, nch, chunk_body, 0)


def _ffn(offs, cnt, xs, W1, b1, W2, b2):
    grid_spec = pltpu.PrefetchScalarGridSpec(
        num_scalar_prefetch=2,
        grid=(NUM_EXPERTS,),
        in_specs=[
            pl.BlockSpec((N_BUF, D_IN), lambda e, offs, cnt: (0, 0)),
            pl.BlockSpec((1, D_IN, D_H), lambda e, offs, cnt: (e, 0, 0)),
            pl.BlockSpec((1, 1, D_H), lambda e, offs, cnt: (e, 0, 0)),
            pl.BlockSpec((1, D_H, D_OUT), lambda e, offs, cnt: (e, 0, 0)),
            pl.BlockSpec((1, 1, D_OUT), lambda e, offs, cnt: (e, 0, 0)),
        ],
        out_specs=pl.BlockSpec((N_BUF, D_OUT), lambda e, offs, cnt: (0, 0)),
    )
    return pl.pallas_call(
        _ffn_body,
        grid_spec=grid_spec,
        out_shape=jax.ShapeDtypeStruct((N_BUF, D_OUT), jnp.float32),
        compiler_params=pltpu.CompilerParams(
            dimension_semantics=("arbitrary",)),
    )(offs, cnt, xs, W1, b1.reshape(NUM_EXPERTS, 1, D_H),
      W2, b2.reshape(NUM_EXPERTS, 1, D_OUT))


def _combine_body(ys_hbm, pos_hbm, wv_hbm, out_hbm,
                  pos_v, wv_v, y0_v, y1_v, sem0, sem1):
    wid = lax.axis_index("s") * NC + lax.axis_index("c")
    base = wid * TPW
    pltpu.sync_copy(pos_hbm.at[wid], pos_v)
    pltpu.sync_copy(wv_hbm.at[wid], wv_v)
    g0 = pltpu.async_copy(ys_hbm.at[pos_v.at[0]], y0_v, sem0)
    g1 = pltpu.async_copy(ys_hbm.at[pos_v.at[1]], y1_v, sem1)
    g0.wait()
    g1.wait()

    def tbody(t, carry):
        w0 = wv_v[0, t, :]
        w1 = wv_v[1, t, :]
        for c in range(D_OUT // LANES):
            sl = pl.ds(c * LANES, LANES)
            y0_v[t, sl] = y0_v[t, sl] * w0 + y1_v[t, sl] * w1
        return carry

    lax.fori_loop(0, TPW, tbody, 0)
    pltpu.sync_copy(y0_v, out_hbm.at[pl.ds(base, TPW)])


def _combine(ys, pos3, wv3):
    mesh = plsc.VectorSubcoreMesh(core_axis_name="c", subcore_axis_name="s")
    fn = pl.kernel(
        _combine_body,
        out_type=jax.ShapeDtypeStruct((T, D_OUT), jnp.float32),
        mesh=mesh,
        scratch_types=[
            pltpu.VMEM((TOP_K, TPW), jnp.int32),
            pltpu.VMEM((TOP_K, TPW, LANES), jnp.float32),
            pltpu.VMEM((TPW, D_OUT), jnp.float32),
            pltpu.VMEM((TPW, D_OUT), jnp.float32),
            pltpu.SemaphoreType.DMA,
            pltpu.SemaphoreType.DMA,
        ],
    )
    return fn(ys, pos3, wv3)


def kernel(x, Wg, bg, W1, b1, W2, b2):
    pos0, pos1, w0, w1, offs, cnt = _routing(x, Wg, bg)

    pos3 = jnp.stack([pos0.reshape(NW, TPW), pos1.reshape(NW, TPW)], axis=1)
    wv3 = jnp.broadcast_to(
        jnp.stack([w0.reshape(NW, TPW), w1.reshape(NW, TPW)], axis=1)[..., None],
        (NW, TOP_K, TPW, LANES))

    xs = _dispatch(x, pos3)
    ys = _ffn(offs.reshape(NUM_EXPERTS), cnt.reshape(NUM_EXPERTS),
              xs, W1, b1, W2, b2)
    return _combine(ys, pos3, wv3)


# speedup vs baseline: 1.1437x; 1.1416x over previous
import functools

import jax
import jax.numpy as jnp
from jax import lax
from jax.experimental import pallas as pl
from jax.experimental.pallas import tpu as pltpu
from jax.experimental.pallas import tpu_sc as plsc

NUM_EXPERTS = 64
TOP_K = 2
D_IN = 768
D_H = 64
D_OUT = 768
T = 2048

TB = 128
N_TB = T // TB
ALIGN = 8
CHUNK = 128
N_BUF = 4736

NC = 2
NS = 16
NW = NC * NS
TPW = T // NW
LANES = 16


def _routing_body(x_ref, wg_ref, bg_ref,
                  pos0_ref, pos1_ref, w0_ref, w1_ref, ce_ref, cb_ref,
                  oh0_s, oh1_s, r0_s, r1_s, w0_s, w1_s, cnt_s):
    i = pl.program_id(0)

    @pl.when(i == 0)
    def _init():
        cnt_s[...] = jnp.zeros((1, NUM_EXPERTS), jnp.float32)

    @pl.when(i < N_TB)
    def _pass_a():
        xb = x_ref[...]
        logits = jnp.dot(xb, wg_ref[...],
                         preferred_element_type=jnp.float32) + bg_ref[...]
        m = jnp.max(logits, axis=1, keepdims=True)
        e = jnp.exp(logits - m)
        w = e / jnp.sum(e, axis=1, keepdims=True)

        lane = lax.broadcasted_iota(jnp.int32, (TB, NUM_EXPERTS), 1)
        m1 = jnp.max(w, axis=1, keepdims=True)
        i1 = jnp.min(jnp.where(w == m1, lane, NUM_EXPERTS), axis=1,
                     keepdims=True)
        oh0 = (lane == i1).astype(jnp.float32)
        wm = jnp.where(lane == i1, -1.0, w)
        m2 = jnp.max(wm, axis=1, keepdims=True)
        i2 = jnp.min(jnp.where(wm == m2, lane, NUM_EXPERTS), axis=1,
                     keepdims=True)
        oh1 = (lane == i2).astype(jnp.float32)

        s = oh0 + oh1
        row = lax.broadcasted_iota(jnp.int32, (TB, TB), 0)
        col = lax.broadcasted_iota(jnp.int32, (TB, TB), 1)
        ltri = (col < row).astype(jnp.float32)
        prefix = jnp.dot(ltri, s, preferred_element_type=jnp.float32)
        carry = cnt_s[...]
        base = carry + prefix
        r0 = jnp.sum(oh0 * base, axis=1, keepdims=True)
        r1 = jnp.sum(oh1 * (base + oh0), axis=1, keepdims=True)
        cnt_s[...] = carry + jnp.sum(s, axis=0, keepdims=True)

        sl = pl.ds(i * TB, TB)
        oh0_s[sl, :] = oh0
        oh1_s[sl, :] = oh1
        r0_s[sl, :] = r0
        r1_s[sl, :] = r1
        w0_s[sl, :] = m1
        w1_s[sl, :] = m2

    @pl.when(i == N_TB)
    def _finalize():
        cnt = cnt_s[...]
        nal = jnp.floor((cnt + (ALIGN - 1)) * (1.0 / ALIGN)) * float(ALIGN)
        erow = lax.broadcasted_iota(jnp.int32, (NUM_EXPERTS, NUM_EXPERTS), 0)
        ecol = lax.broadcasted_iota(jnp.int32, (NUM_EXPERTS, NUM_EXPERTS), 1)
        nal_b = jnp.broadcast_to(nal, (NUM_EXPERTS, NUM_EXPERTS))
        offs_c = jnp.sum(jnp.where(ecol < erow, nal_b, 0.0),
                         axis=1, keepdims=True)
        offs_row = jnp.sum(
            jnp.where(erow == ecol,
                      jnp.broadcast_to(offs_c, (NUM_EXPERTS, NUM_EXPERTS)),
                      0.0), axis=0, keepdims=True)

        pos0 = r0_s[...] + jnp.sum(oh0_s[...] * offs_row, axis=1,
                                   keepdims=True)
        pos1 = r1_s[...] + jnp.sum(oh1_s[...] * offs_row, axis=1,
                                   keepdims=True)
        pos0_ref[...] = pos0.astype(jnp.int32)
        pos1_ref[...] = pos1.astype(jnp.int32)
        w0_ref[...] = w0_s[...]
        w1_ref[...] = w1_s[...]
        ce_ref[...] = offs_row.astype(jnp.int32)
        cb_ref[...] = cnt.astype(jnp.int32)


def _routing(x, Wg, bg):
    f32 = jnp.float32
    i32 = jnp.int32
    return pl.pallas_call(
        _routing_body,
        grid=(N_TB + 1,),
        in_specs=[
            pl.BlockSpec((TB, D_IN), lambda i: (jnp.minimum(i, N_TB - 1), 0)),
            pl.BlockSpec((D_IN, NUM_EXPERTS), lambda i: (0, 0)),
            pl.BlockSpec((1, NUM_EXPERTS), lambda i: (0, 0)),
        ],
        out_specs=[
            pl.BlockSpec((T, 1), lambda i: (0, 0)),
            pl.BlockSpec((T, 1), lambda i: (0, 0)),
            pl.BlockSpec((T, 1), lambda i: (0, 0)),
            pl.BlockSpec((T, 1), lambda i: (0, 0)),
            pl.BlockSpec((1, NUM_EXPERTS), lambda i: (0, 0)),
            pl.BlockSpec((1, NUM_EXPERTS), lambda i: (0, 0)),
        ],
        out_shape=[
            jax.ShapeDtypeStruct((T, 1), i32),
            jax.ShapeDtypeStruct((T, 1), i32),
            jax.ShapeDtypeStruct((T, 1), f32),
            jax.ShapeDtypeStruct((T, 1), f32),
            jax.ShapeDtypeStruct((1, NUM_EXPERTS), i32),
            jax.ShapeDtypeStruct((1, NUM_EXPERTS), i32),
        ],
        scratch_shapes=[
            pltpu.VMEM((T, NUM_EXPERTS), f32),
            pltpu.VMEM((T, NUM_EXPERTS), f32),
            pltpu.VMEM((T, 1), f32),
            pltpu.VMEM((T, 1), f32),
            pltpu.VMEM((T, 1), f32),
            pltpu.VMEM((T, 1), f32),
            pltpu.VMEM((1, NUM_EXPERTS), f32),
        ],
        compiler_params=pltpu.CompilerParams(
            dimension_semantics=("arbitrary",)),
    )(x, Wg, bg.reshape(1, NUM_EXPERTS))


def _dispatch_body(x_hbm, pos_hbm, xs_hbm, pos_v, rows_v, sem0, sem1):
    wid = lax.axis_index("s") * NC + lax.axis_index("c")
    base = wid * TPW
    pltpu.sync_copy(pos_hbm.at[wid], pos_v)
    pltpu.sync_copy(x_hbm.at[pl.ds(base, TPW)], rows_v)
    c0 = pltpu.async_copy(rows_v, xs_hbm.at[pos_v.at[0]], sem0)
    c1 = pltpu.async_copy(rows_v, xs_hbm.at[pos_v.at[1]], sem1)
    c0.wait()
    c1.wait()


def _dispatch(x, pos3):
    mesh = plsc.VectorSubcoreMesh(core_axis_name="c", subcore_axis_name="s")
    fn = pl.kernel(
        _dispatch_body,
        out_type=jax.ShapeDtypeStruct((N_BUF, D_IN), jnp.float32),
        mesh=mesh,
        scratch_types=[
            pltpu.VMEM((TOP_K, TPW), jnp.int32),
            pltpu.VMEM((TPW, D_IN), jnp.float32),
            pltpu.SemaphoreType.DMA,
            pltpu.SemaphoreType.DMA,
        ],
    )
    return fn(x, pos3)


EPG = 8


def _ffn_body(offs_ref, cnt_ref, xs_ref, w1_ref, b1_ref, w2_ref, b2_ref,
              ys_ref):
    g = pl.program_id(0)
    for ee in range(EPG):
        e = g * EPG + ee
        off = offs_ref[e]
        nch = (cnt_ref[e] + (CHUNK - 1)) // CHUNK
        w1 = w1_ref[ee]
        b1v = b1_ref[ee]
        w2 = w2_ref[ee]
        b2v = b2_ref[ee]

        def chunk_body(j, carry, off=off, w1=w1, b1v=b1v, w2=w2, b2v=b2v):
            sl = pl.ds(pl.multiple_of(off + j * CHUNK, ALIGN), CHUNK)
            xb = xs_ref[sl, :]
            h = jnp.maximum(
                jnp.dot(xb, w1, preferred_element_type=jnp.float32) + b1v,
                0.0)
            y = jnp.dot(h, w2, preferred_element_type=jnp.float32) + b2v
            ys_ref[sl, :] = y
            return carry

        lax.fori_loop(0, nch, chunk_body, 0)


def _ffn(offs, cnt, xs, W1, b1, W2, b2):
    grid_spec = pltpu.PrefetchScalarGridSpec(
        num_scalar_prefetch=2,
        grid=(NUM_EXPERTS // EPG,),
        in_specs=[
            pl.BlockSpec((N_BUF, D_IN), lambda g, offs, cnt: (0, 0)),
            pl.BlockSpec((EPG, D_IN, D_H), lambda g, offs, cnt: (g, 0, 0)),
            pl.BlockSpec((EPG, 1, D_H), lambda g, offs, cnt: (g, 0, 0)),
            pl.BlockSpec((EPG, D_H, D_OUT), lambda g, offs, cnt: (g, 0, 0)),
            pl.BlockSpec((EPG, 1, D_OUT), lambda g, offs, cnt: (g, 0, 0)),
        ],
        out_specs=pl.BlockSpec((N_BUF, D_OUT), lambda g, offs, cnt: (0, 0)),
    )
    return pl.pallas_call(
        _ffn_body,
        grid_spec=grid_spec,
        out_shape=jax.ShapeDtypeStruct((N_BUF, D_OUT), jnp.float32),
        compiler_params=pltpu.CompilerParams(
            dimension_semantics=("arbitrary",)),
    )(offs, cnt, xs, W1, b1.reshape(NUM_EXPERTS, 1, D_H),
      W2, b2.reshape(NUM_EXPERTS, 1, D_OUT))


def _combine_body(ys_hbm, pos_hbm, wv_hbm, out_hbm,
                  pos_v, wv_v, y0_v, y1_v, sem0, sem1):
    wid = lax.axis_index("s") * NC + lax.axis_index("c")
    base = wid * TPW
    pltpu.sync_copy(pos_hbm.at[wid], pos_v)
    pltpu.sync_copy(wv_hbm.at[wid], wv_v)
    g0 = pltpu.async_copy(ys_hbm.at[pos_v.at[0]], y0_v, sem0)
    g1 = pltpu.async_copy(ys_hbm.at[pos_v.at[1]], y1_v, sem1)
    g0.wait()
    g1.wait()

    def tbody(t, carry):
        w0 = wv_v[0, t, :]
        w1 = wv_v[1, t, :]
        for c in range(D_OUT // LANES):
            sl = pl.ds(c * LANES, LANES)
            y0_v[t, sl] = y0_v[t, sl] * w0 + y1_v[t, sl] * w1
        return carry

    lax.fori_loop(0, TPW, tbody, 0)
    pltpu.sync_copy(y0_v, out_hbm.at[pl.ds(base, TPW)])


def _combine(ys, pos3, wv3):
    mesh = plsc.VectorSubcoreMesh(core_axis_name="c", subcore_axis_name="s")
    fn = pl.kernel(
        _combine_body,
        out_type=jax.ShapeDtypeStruct((T, D_OUT), jnp.float32),
        mesh=mesh,
        scratch_types=[
            pltpu.VMEM((TOP_K, TPW), jnp.int32),
            pltpu.VMEM((TOP_K, TPW, LANES), jnp.float32),
            pltpu.VMEM((TPW, D_OUT), jnp.float32),
            pltpu.VMEM((TPW, D_OUT), jnp.float32),
            pltpu.SemaphoreType.DMA,
            pltpu.SemaphoreType.DMA,
        ],
    )
    return fn(ys, pos3, wv3)


def kernel(x, Wg, bg, W1, b1, W2, b2):
    pos0, pos1, w0, w1, offs, cnt = _routing(x, Wg, bg)

    pos3 = jnp.stack([pos0.reshape(NW, TPW), pos1.reshape(NW, TPW)], axis=1)
    wv3 = jnp.broadcast_to(
        jnp.stack([w0.reshape(NW, TPW), w1.reshape(NW, TPW)], axis=1)[..., None],
        (NW, TOP_K, TPW, LANES))

    xs = _dispatch(x, pos3)
    ys = _ffn(offs.reshape(NUM_EXPERTS), cnt.reshape(NUM_EXPERTS),
              xs, W1, b1, W2, b2)
    return _combine(ys, pos3, wv3)
